# baseline (speedup 1.0000x reference)
import functools

import jax
import jax.numpy as jnp
from jax import lax
from jax.experimental import pallas as pl
from jax.experimental.pallas import tpu as pltpu
from jax.experimental.pallas import tpu_sc as plsc

N = 10000
D = 256
E = 160000

NC = 2
NS = 16
LANES = 16

NPAD = 10240
HALF = NPAD // NC
ROWS_PER_TILE = HALF // NS
EPAD = 163840
K = 128
ZROW = N
DUMMY = HALF


_sc_mesh = plsc.VectorSubcoreMesh(core_axis_name="c", subcore_axis_name="s")


@functools.partial(
    pl.kernel,
    out_type=jax.ShapeDtypeStruct((NC, NPAD), jnp.float32),
    mesh=_sc_mesh,
    scratch_types=[
        pltpu.VMEM((5120,), jnp.int32),
        pltpu.VMEM((K,), jnp.float32),
        pltpu.VMEM((K,), jnp.int32),
        pltpu.VMEM((NPAD // NS,), jnp.float32),
        pltpu.VMEM_SHARED((NPAD,), jnp.float32),
    ],
)
def _degree_kernel(dst_hbm, deg_out, dst_v, ones_v, idx_v, zero_v, deg_sp):
    c = lax.axis_index("c")
    s = lax.axis_index("s")
    wid = c * NS + s

    def _fill(i, _):
        zero_v[pl.ds(i * LANES, LANES)] = jnp.zeros((LANES,), jnp.float32)
        return 0

    lax.fori_loop(0, (NPAD // NS) // LANES, _fill, 0)
    pltpu.sync_copy(zero_v, deg_sp.at[pl.ds(s * (NPAD // NS), NPAD // NS)])

    def _fill1(i, _):
        ones_v[pl.ds(i * LANES, LANES)] = jnp.ones((LANES,), jnp.float32)
        return 0

    lax.fori_loop(0, K // LANES, _fill1, 0)
    pltpu.sync_copy(dst_hbm.at[wid], dst_v)
    plsc.subcore_barrier()

    def _chunk(j, _):
        def _cp(t, _):
            idx_v[pl.ds(t * LANES, LANES)] = dst_v[pl.ds(j * K + t * LANES, LANES)]
            return 0

        lax.fori_loop(0, K // LANES, _cp, 0)
        pltpu.sync_copy(ones_v, deg_sp.at[idx_v], add=True)
        return 0

    lax.fori_loop(0, 5120 // K, _chunk, 0)
    plsc.subcore_barrier()
    pltpu.sync_copy(deg_sp.at[pl.ds(s * (NPAD // NS), NPAD // NS)],
                    deg_out.at[c, pl.ds(s * (NPAD // NS), NPAD // NS)])


SEG = 1024
EPT = EPAD // NS
KE = 64
CAP = 6144
TRASH = 2048
_STG = CAP + TRASH
_ACC_WORDS = (HALF + NS) * D


@functools.partial(
    pl.kernel,
    out_type=jax.ShapeDtypeStruct((NPAD * D,), jnp.float32),
    mesh=_sc_mesh,
    scratch_types=[
        pltpu.VMEM((SEG,), jnp.int32),
        pltpu.VMEM((SEG,), jnp.int32),
        pltpu.VMEM((KE,), jnp.int32),
        pltpu.VMEM((KE,), jnp.int32),
        pltpu.VMEM((KE, D), jnp.float32),
        pltpu.VMEM((2 * KE, K), jnp.int32),
        pltpu.VMEM((K,), jnp.int32),
        pltpu.VMEM((K,), jnp.int32),
        pltpu.VMEM((SEG,), jnp.int32),
        pltpu.VMEM_SHARED((_ACC_WORDS,), jnp.float32),
        pltpu.VMEM_SHARED((NS * _STG,), jnp.int32),
        pltpu.SemaphoreType.DMA,
        pltpu.SemaphoreType.DMA,
    ],
)
def _agg_kernel(zf_hbm, z2_hbm, src_hbm, dst_hbm, aggf_out,
                sseg_v, dseg_v, gidx_v, wb_v, rows_v, aidx_v, sidx_v, sval_v,
                zero_v, acc_sp, stg_sp, sem, gsem):
    c = lax.axis_index("c")
    s = lax.axis_index("s")
    nbase = c * HALF
    wslice = ROWS_PER_TILE * D
    ebase = s * EPT
    sbase = s * _STG

    pltpu.sync_copy(zf_hbm.at[pl.ds((nbase + s * ROWS_PER_TILE) * D, wslice)],
                    acc_sp.at[pl.ds(s * ROWS_PER_TILE * D, wslice)])

    iota = lax.broadcasted_iota(jnp.int32, (LANES,), 0)
    zeros_i = jnp.zeros((LANES,), jnp.int32)
    ones_i = jnp.ones((LANES,), jnp.int32)
    last_lane = jnp.full((LANES,), LANES - 1, jnp.int32)
    capv = jnp.full((LANES,), CAP, jnp.int32)

    def _z(i, _):
        zero_v[pl.ds(i * LANES, LANES)] = zeros_i
        return 0

    lax.fori_loop(0, SEG // LANES, _z, 0)

    def _z2(k, _):
        pltpu.sync_copy(zero_v, stg_sp.at[pl.ds(sbase + k * SEG, SEG)])
        return 0

    lax.fori_loop(0, _STG // SEG, _z2, 0)
    plsc.subcore_barrier()

    def _scan_seg(g, off_v):
        pltpu.sync_copy(src_hbm.at[pl.ds(ebase + g * SEG, SEG)], sseg_v)
        pltpu.sync_copy(dst_hbm.at[pl.ds(ebase + g * SEG, SEG)], dseg_v)

        def _cchunk(jj, off_v):
            def _vreg(t, off_v):
                sl = pl.ds(jj * K + t * LANES, LANES)
                d16 = dseg_v[sl]
                s16 = sseg_v[sl]
                dl = d16 - nbase
                m = jnp.logical_and(dl >= 0, dl < HALF)
                cs = jnp.where(m, ones_i, zeros_i)
                for kk in (1, 2, 4, 8):
                    sh = cs[jnp.maximum(iota - kk, zeros_i)]
                    cs = cs + jnp.where(iota >= kk, sh, zeros_i)
                pos = off_v + cs - 1
                m2 = jnp.logical_and(m, pos < capv)
                lane = jj * K + t * LANES
                sidx_v[pl.ds(t * LANES, LANES)] = sbase + jnp.where(
                    m2, pos, CAP + lane + iota)
                sval_v[pl.ds(t * LANES, LANES)] = jnp.where(
                    m2, jnp.bitwise_or(jnp.left_shift(dl + 1, 14), s16),
                    zeros_i)
                return off_v + cs[last_lane]

            off_v = lax.fori_loop(0, K // LANES, _vreg, off_v)
            pltpu.sync_copy(sval_v, stg_sp.at[sidx_v], add=True)
            return off_v

        return lax.fori_loop(0, SEG // K, _cchunk, off_v)

    lax.fori_loop(0, EPT // SEG, _scan_seg, zeros_i)

    def _drain_seg(g, _):
        pltpu.sync_copy(stg_sp.at[pl.ds(sbase + g * SEG, SEG)], sseg_v)

        def _chunk(j, _):
            def _prep(t, _):
                pk = sseg_v[pl.ds(j * KE + t * LANES, LANES)]
                srcv = jnp.bitwise_and(pk, jnp.full((LANES,), 16383, jnp.int32))
                dlp = lax.shift_right_logical(pk, 14)
                valid = dlp > 0
                gidx_v[pl.ds(t * LANES, LANES)] = jnp.where(
                    valid, srcv, jnp.full((LANES,), ZROW, jnp.int32))
                wb_v[pl.ds(t * LANES, LANES)] = jnp.where(
                    valid, dlp - 1, jnp.full((LANES,), HALF, jnp.int32) + s) * D
                return 0

            lax.fori_loop(0, KE // LANES, _prep, 0)
            gd = pltpu.async_copy(z2_hbm.at[gidx_v], rows_v, gsem)

            def _build(e, _):
                wv = wb_v[pl.ds((e // LANES) * LANES, LANES)]
                wb_e = wv[jnp.full((LANES,), e % LANES, jnp.int32)]
                for q in range(D // LANES):
                    aidx_v[2 * e + q // 8, pl.ds((q % 8) * LANES, LANES)] = (
                        wb_e + (q * LANES) + iota)
                return 0

            lax.fori_loop(0, KE, _build, 0)
            gd.wait()

            def _fire(e, _):
                pltpu.async_copy(rows_v.at[e, pl.ds(0, K)],
                                 acc_sp.at[aidx_v.at[2 * e]], sem, add=True)
                pltpu.async_copy(rows_v.at[e, pl.ds(K, K)],
                                 acc_sp.at[aidx_v.at[2 * e + 1]], sem, add=True)
                return 0

            lax.fori_loop(0, KE, _fire, 0)
            pltpu.make_async_copy(z2_hbm.at[pl.ds(0, KE)], rows_v, sem).wait()
            return 0

        lax.fori_loop(0, SEG // KE, _chunk, 0)
        return 0

    lax.fori_loop(0, CAP // SEG, _drain_seg, 0)
    plsc.subcore_barrier()
    pltpu.sync_copy(acc_sp.at[pl.ds(s * ROWS_PER_TILE * D, wslice)],
                    aggf_out.at[pl.ds((nbase + s * ROWS_PER_TILE) * D, wslice)])


def _prep_body(x_ref, p_ref, deg_ref, score_ref, dinv_ref, z_ref):
    x = x_ref[...]
    p = p_ref[...]
    pn = jnp.sqrt(jnp.sum(p * p))
    sc = jnp.tanh(jax.lax.dot(x, p, preferred_element_type=jnp.float32) / pn)
    rows = lax.broadcasted_iota(jnp.int32, (NPAD, 1), 0)
    score_ref[...] = jnp.where(rows < N, sc, jnp.float32(-2.0))
    dg = deg_ref[...]
    deg = dg[:, 0:1] + dg[:, 1:2] + 1.0
    dinv = lax.rsqrt(deg)
    dinv_ref[...] = dinv
    z_ref[...] = x * dinv


def _prep(x_pad, p2, deg_t):
    return pl.pallas_call(
        _prep_body,
        out_shape=(
            jax.ShapeDtypeStruct((NPAD, 1), jnp.float32),
            jax.ShapeDtypeStruct((NPAD, 1), jnp.float32),
            jax.ShapeDtypeStruct((NPAD, D), jnp.float32),
        ),
    )(x_pad, p2, deg_t)


def _evolve_body(s_ref, x_ref, wih_ref, whh_ref, bih_ref, bhh_ref, w0_ref,
                 w_ref):
    S0 = s_ref[...]
    x = x_ref[...]
    flat = (lax.broadcasted_iota(jnp.int32, (80, 128), 0) * 128
            + lax.broadcasted_iota(jnp.int32, (80, 128), 1))
    lane = lax.broadcasted_iota(jnp.int32, (1, NPAD), 1)
    rowio = lax.broadcasted_iota(jnp.int32, (D, 1), 0)

    def _it(i, carry):
        S, Xt = carry
        m = jnp.max(S)
        idx = jnp.min(jnp.where(S == m, flat, jnp.int32(1 << 30)))
        onehot = (lane == idx).astype(jnp.float32)
        row = lax.dot_general(onehot, x, (((1,), (0,)), ((), ())),
                              preferred_element_type=jnp.float32) * m
        Xt = jnp.where(rowio == i, row, Xt)
        S = jnp.where(flat == idx, jnp.float32(-3e38), S)
        return S, Xt

    _, Xt = lax.fori_loop(0, D, _it, (S0, jnp.zeros((D, D), jnp.float32)))

    w0 = w0_ref[...]
    gi = lax.dot_general(Xt, wih_ref[...], (((1,), (1,)), ((), ())),
                         preferred_element_type=jnp.float32, precision=lax.Precision.HIGHEST) + bih_ref[...]
    gh = lax.dot_general(w0, whh_ref[...], (((1,), (1,)), ((), ())),
                         preferred_element_type=jnp.float32, precision=lax.Precision.HIGHEST) + bhh_ref[...]
    r = jax.nn.sigmoid(gi[:, 0:D] + gh[:, 0:D])
    z = jax.nn.sigmoid(gi[:, D:2 * D] + gh[:, D:2 * D])
    n = jnp.tanh(gi[:, 2 * D:3 * D] + r * gh[:, 2 * D:3 * D])
    w_ref[...] = (1.0 - z) * n + z * w0


def _evolve(score80, x_pad, wih, whh, bih2, bhh2, w0):
    return pl.pallas_call(
        _evolve_body,
        out_shape=jax.ShapeDtypeStruct((D, D), jnp.float32),
    )(score80, x_pad, wih, whh, bih2, bhh2, w0)


_BLK = 512


def _final_body(a_ref, dv_ref, w_ref, lw_ref, lb_ref, out_ref):
    t = jax.lax.dot(a_ref[...], w_ref[...],
                    preferred_element_type=jnp.float32, precision=lax.Precision.HIGHEST) * dv_ref[...]
    t = jnp.maximum(t, 0.0)
    out_ref[...] = lax.dot_general(t, lw_ref[...], (((1,), (1,)), ((), ())),
                                   preferred_element_type=jnp.float32, precision=lax.Precision.HIGHEST) + lb_ref[...]


def _final(agg, dinv, w, lin_w, lb2):
    return pl.pallas_call(
        _final_body,
        grid=(NPAD // _BLK,),
        in_specs=[
            pl.BlockSpec((_BLK, D), lambda i: (i, 0)),
            pl.BlockSpec((_BLK, 1), lambda i: (i, 0)),
            pl.BlockSpec((D, D), lambda i: (0, 0)),
            pl.BlockSpec((D, D), lambda i: (0, 0)),
            pl.BlockSpec((1, D), lambda i: (0, 0)),
        ],
        out_specs=pl.BlockSpec((_BLK, D), lambda i: (i, 0)),
        out_shape=jax.ShapeDtypeStruct((NPAD, D), jnp.float32),
    )(agg, dinv, w, lin_w, lb2)


def kernel(x, edge_index, pool_p, gru_W_ih, gru_W_hh, gru_b_ih, gru_b_hh,
           W0, lin_W, lin_b):
    src = edge_index[0]
    dst = edge_index[1]
    npad_rows = NPAD - N
    x_pad = jnp.concatenate(
        [x, jnp.zeros((npad_rows, D), jnp.float32)], axis=0)
    epad = EPAD - E
    src_p = jnp.concatenate([src, jnp.full((epad,), ZROW, jnp.int32)])
    dst_p = jnp.concatenate([dst, jnp.full((epad,), NPAD - 1, jnp.int32)])

    deg_parts = _degree_kernel(dst_p.reshape(NC * NS, EPAD // (NC * NS)))
    score, dinv, z = _prep(x_pad, pool_p.reshape(D, 1),
                           deg_parts.T.reshape(NPAD, NC))
    w = _evolve(score.reshape(80, 128), x_pad, gru_W_ih, gru_W_hh,
                gru_b_ih.reshape(1, 3 * D), gru_b_hh.reshape(1, 3 * D), W0)
    aggf = _agg_kernel(z.reshape(NPAD * D), z, src_p, dst_p)
    h = _final(aggf.reshape(NPAD, D), dinv, w, lin_W, lin_b.reshape(1, D))
    return h[:N]


# speedup vs baseline: 1.5376x; 1.5376x over previous
import functools

import jax
import jax.numpy as jnp
from jax import lax
from jax.experimental import pallas as pl
from jax.experimental.pallas import tpu as pltpu
from jax.experimental.pallas import tpu_sc as plsc

N = 10000
D = 256
E = 160000

NC = 2
NS = 16
LANES = 16

NPAD = 10240
HALF = NPAD // NC
ROWS_PER_TILE = HALF // NS
EPAD = 163840
K = 128
ZROW = N
DUMMY = HALF


_sc_mesh = plsc.VectorSubcoreMesh(core_axis_name="c", subcore_axis_name="s")


@functools.partial(
    pl.kernel,
    out_type=jax.ShapeDtypeStruct((NC, NPAD), jnp.float32),
    mesh=_sc_mesh,
    scratch_types=[
        pltpu.VMEM((5120,), jnp.int32),
        pltpu.VMEM((K,), jnp.float32),
        pltpu.VMEM((K,), jnp.int32),
        pltpu.VMEM((NPAD // NS,), jnp.float32),
        pltpu.VMEM_SHARED((NPAD,), jnp.float32),
    ],
)
def _degree_kernel(dst_hbm, deg_out, dst_v, ones_v, idx_v, zero_v, deg_sp):
    c = lax.axis_index("c")
    s = lax.axis_index("s")
    wid = c * NS + s

    def _fill(i, _):
        zero_v[pl.ds(i * LANES, LANES)] = jnp.zeros((LANES,), jnp.float32)
        return 0

    lax.fori_loop(0, (NPAD // NS) // LANES, _fill, 0)
    pltpu.sync_copy(zero_v, deg_sp.at[pl.ds(s * (NPAD // NS), NPAD // NS)])

    def _fill1(i, _):
        ones_v[pl.ds(i * LANES, LANES)] = jnp.ones((LANES,), jnp.float32)
        return 0

    lax.fori_loop(0, K // LANES, _fill1, 0)
    pltpu.sync_copy(dst_hbm.at[wid], dst_v)
    plsc.subcore_barrier()

    def _chunk(j, _):
        def _cp(t, _):
            idx_v[pl.ds(t * LANES, LANES)] = dst_v[pl.ds(j * K + t * LANES, LANES)]
            return 0

        lax.fori_loop(0, K // LANES, _cp, 0)
        pltpu.sync_copy(ones_v, deg_sp.at[idx_v], add=True)
        return 0

    lax.fori_loop(0, 5120 // K, _chunk, 0)
    plsc.subcore_barrier()
    pltpu.sync_copy(deg_sp.at[pl.ds(s * (NPAD // NS), NPAD // NS)],
                    deg_out.at[c, pl.ds(s * (NPAD // NS), NPAD // NS)])


SEG = 1024
EPT = EPAD // NS
KE = 64
_ACC_WORDS = (HALF + NS) * D


@functools.partial(
    pl.kernel,
    out_type=jax.ShapeDtypeStruct((NPAD * D,), jnp.float32),
    mesh=_sc_mesh,
    scratch_types=[
        pltpu.VMEM((SEG,), jnp.int32),
        pltpu.VMEM((SEG,), jnp.int32),
        pltpu.VMEM((KE,), jnp.int32),
        pltpu.VMEM((KE,), jnp.int32),
        pltpu.VMEM((KE, D), jnp.float32),
        pltpu.VMEM((2 * KE, K), jnp.int32),
        pltpu.VMEM_SHARED((_ACC_WORDS,), jnp.float32),
        pltpu.SemaphoreType.DMA,
        pltpu.SemaphoreType.DMA,
    ],
)
def _agg_kernel(zf_hbm, z2_hbm, src_hbm, dst_hbm, aggf_out,
                sseg_v, dseg_v, gidx_v, wb_v, rows_v, aidx_v, acc_sp, sem,
                gsem):
    c = lax.axis_index("c")
    s = lax.axis_index("s")
    nbase = c * HALF
    wslice = ROWS_PER_TILE * D
    ebase = s * EPT

    pltpu.sync_copy(zf_hbm.at[pl.ds((nbase + s * ROWS_PER_TILE) * D, wslice)],
                    acc_sp.at[pl.ds(s * ROWS_PER_TILE * D, wslice)])
    plsc.subcore_barrier()

    iota = lax.broadcasted_iota(jnp.int32, (LANES,), 0)

    def _segment(g, _):
        pltpu.sync_copy(src_hbm.at[pl.ds(ebase + g * SEG, SEG)], sseg_v)
        pltpu.sync_copy(dst_hbm.at[pl.ds(ebase + g * SEG, SEG)], dseg_v)

        def _chunk(j, _):
            def _prep(t, _):
                sl = pl.ds(j * KE + t * LANES, LANES)
                gidx_v[pl.ds(t * LANES, LANES)] = sseg_v[sl]
                d16 = dseg_v[sl]
                dl = d16 - nbase
                m = jnp.logical_and(dl >= 0, dl < HALF)
                wb_v[pl.ds(t * LANES, LANES)] = jnp.where(
                    m, dl, jnp.full((LANES,), HALF, jnp.int32) + s) * D
                return 0

            lax.fori_loop(0, KE // LANES, _prep, 0)
            gd = pltpu.async_copy(z2_hbm.at[gidx_v], rows_v, gsem)

            def _build(e, _):
                wv = wb_v[pl.ds((e // LANES) * LANES, LANES)]
                wb_e = wv[jnp.full((LANES,), e % LANES, jnp.int32)]
                for q in range(D // LANES):
                    aidx_v[2 * e + q // 8, pl.ds((q % 8) * LANES, LANES)] = (
                        wb_e + (q * LANES) + iota)
                return 0

            lax.fori_loop(0, KE, _build, 0)
            gd.wait()

            def _fire(e, _):
                pltpu.async_copy(rows_v.at[e, pl.ds(0, K)],
                                 acc_sp.at[aidx_v.at[2 * e]], sem, add=True)
                pltpu.async_copy(rows_v.at[e, pl.ds(K, K)],
                                 acc_sp.at[aidx_v.at[2 * e + 1]], sem, add=True)
                return 0

            lax.fori_loop(0, KE, _fire, 0)
            pltpu.make_async_copy(z2_hbm.at[pl.ds(0, KE)], rows_v, sem).wait()
            return 0

        lax.fori_loop(0, SEG // KE, _chunk, 0)
        return 0

    lax.fori_loop(0, EPT // SEG, _segment, 0)
    plsc.subcore_barrier()
    pltpu.sync_copy(acc_sp.at[pl.ds(s * ROWS_PER_TILE * D, wslice)],
                    aggf_out.at[pl.ds((nbase + s * ROWS_PER_TILE) * D, wslice)])


def _prep_body(x_ref, p_ref, deg_ref, score_ref, dinv_ref, z_ref):
    x = x_ref[...]
    p = p_ref[...]
    pn = jnp.sqrt(jnp.sum(p * p))
    sc = jnp.tanh(jax.lax.dot(x, p, preferred_element_type=jnp.float32) / pn)
    rows = lax.broadcasted_iota(jnp.int32, (NPAD, 1), 0)
    score_ref[...] = jnp.where(rows < N, sc, jnp.float32(-2.0))
    dg = deg_ref[...]
    deg = dg[:, 0:1] + dg[:, 1:2] + 1.0
    dinv = lax.rsqrt(deg)
    dinv_ref[...] = dinv
    z_ref[...] = x * dinv


def _prep(x_pad, p2, deg_t):
    return pl.pallas_call(
        _prep_body,
        out_shape=(
            jax.ShapeDtypeStruct((NPAD, 1), jnp.float32),
            jax.ShapeDtypeStruct((NPAD, 1), jnp.float32),
            jax.ShapeDtypeStruct((NPAD, D), jnp.float32),
        ),
    )(x_pad, p2, deg_t)


def _evolve_body(s_ref, x_ref, wih_ref, whh_ref, bih_ref, bhh_ref, w0_ref,
                 w_ref):
    S0 = s_ref[...]
    x = x_ref[...]
    flat = (lax.broadcasted_iota(jnp.int32, (80, 128), 0) * 128
            + lax.broadcasted_iota(jnp.int32, (80, 128), 1))
    lane = lax.broadcasted_iota(jnp.int32, (1, NPAD), 1)
    rowio = lax.broadcasted_iota(jnp.int32, (D, 1), 0)

    def _it(i, carry):
        S, Xt = carry
        m = jnp.max(S)
        idx = jnp.min(jnp.where(S == m, flat, jnp.int32(1 << 30)))
        onehot = (lane == idx).astype(jnp.float32)
        row = lax.dot_general(onehot, x, (((1,), (0,)), ((), ())),
                              preferred_element_type=jnp.float32) * m
        Xt = jnp.where(rowio == i, row, Xt)
        S = jnp.where(flat == idx, jnp.float32(-3e38), S)
        return S, Xt

    _, Xt = lax.fori_loop(0, D, _it, (S0, jnp.zeros((D, D), jnp.float32)))

    w0 = w0_ref[...]
    gi = lax.dot_general(Xt, wih_ref[...], (((1,), (1,)), ((), ())),
                         preferred_element_type=jnp.float32, precision=lax.Precision.HIGHEST) + bih_ref[...]
    gh = lax.dot_general(w0, whh_ref[...], (((1,), (1,)), ((), ())),
                         preferred_element_type=jnp.float32, precision=lax.Precision.HIGHEST) + bhh_ref[...]
    r = jax.nn.sigmoid(gi[:, 0:D] + gh[:, 0:D])
    z = jax.nn.sigmoid(gi[:, D:2 * D] + gh[:, D:2 * D])
    n = jnp.tanh(gi[:, 2 * D:3 * D] + r * gh[:, 2 * D:3 * D])
    w_ref[...] = (1.0 - z) * n + z * w0


def _evolve(score80, x_pad, wih, whh, bih2, bhh2, w0):
    return pl.pallas_call(
        _evolve_body,
        out_shape=jax.ShapeDtypeStruct((D, D), jnp.float32),
    )(score80, x_pad, wih, whh, bih2, bhh2, w0)


_BLK = 512


def _final_body(a_ref, dv_ref, w_ref, lw_ref, lb_ref, out_ref):
    t = jax.lax.dot(a_ref[...], w_ref[...],
                    preferred_element_type=jnp.float32, precision=lax.Precision.HIGHEST) * dv_ref[...]
    t = jnp.maximum(t, 0.0)
    out_ref[...] = lax.dot_general(t, lw_ref[...], (((1,), (1,)), ((), ())),
                                   preferred_element_type=jnp.float32, precision=lax.Precision.HIGHEST) + lb_ref[...]


def _final(agg, dinv, w, lin_w, lb2):
    return pl.pallas_call(
        _final_body,
        grid=(NPAD // _BLK,),
        in_specs=[
            pl.BlockSpec((_BLK, D), lambda i: (i, 0)),
            pl.BlockSpec((_BLK, 1), lambda i: (i, 0)),
            pl.BlockSpec((D, D), lambda i: (0, 0)),
            pl.BlockSpec((D, D), lambda i: (0, 0)),
            pl.BlockSpec((1, D), lambda i: (0, 0)),
        ],
        out_specs=pl.BlockSpec((_BLK, D), lambda i: (i, 0)),
        out_shape=jax.ShapeDtypeStruct((NPAD, D), jnp.float32),
    )(agg, dinv, w, lin_w, lb2)


def kernel(x, edge_index, pool_p, gru_W_ih, gru_W_hh, gru_b_ih, gru_b_hh,
           W0, lin_W, lin_b):
    src = edge_index[0]
    dst = edge_index[1]
    npad_rows = NPAD - N
    x_pad = jnp.concatenate(
        [x, jnp.zeros((npad_rows, D), jnp.float32)], axis=0)
    epad = EPAD - E
    src_p = jnp.concatenate([src, jnp.full((epad,), ZROW, jnp.int32)])
    dst_p = jnp.concatenate([dst, jnp.full((epad,), NPAD - 1, jnp.int32)])

    deg_parts = _degree_kernel(dst_p.reshape(NC * NS, EPAD // (NC * NS)))
    score, dinv, z = _prep(x_pad, pool_p.reshape(D, 1),
                           deg_parts.T.reshape(NPAD, NC))
    w = _evolve(score.reshape(80, 128), x_pad, gru_W_ih, gru_W_hh,
                gru_b_ih.reshape(1, 3 * D), gru_b_hh.reshape(1, 3 * D), W0)
    aggf = _agg_kernel(z.reshape(NPAD * D), z, src_p, dst_p)
    h = _final(aggf.reshape(NPAD, D), dinv, w, lin_W, lin_b.reshape(1, D))
    return h[:N]


# speedup vs baseline: 1.6908x; 1.0996x over previous
import functools

import jax
import jax.numpy as jnp
from jax import lax
from jax.experimental import pallas as pl
from jax.experimental.pallas import tpu as pltpu
from jax.experimental.pallas import tpu_sc as plsc

N = 10000
D = 256
E = 160000

NC = 2
NS = 16
LANES = 16

NPAD = 10240
HALF = NPAD // NC
ROWS_PER_TILE = HALF // NS
EPAD = 163840
K = 128
ZROW = N
DUMMY = HALF


_sc_mesh = plsc.VectorSubcoreMesh(core_axis_name="c", subcore_axis_name="s")


@functools.partial(
    pl.kernel,
    out_type=jax.ShapeDtypeStruct((NC, NPAD), jnp.float32),
    mesh=_sc_mesh,
    scratch_types=[
        pltpu.VMEM((5120,), jnp.int32),
        pltpu.VMEM((K,), jnp.float32),
        pltpu.VMEM((K,), jnp.int32),
        pltpu.VMEM((NPAD // NS,), jnp.float32),
        pltpu.VMEM_SHARED((NPAD,), jnp.float32),
    ],
)
def _degree_kernel(dst_hbm, deg_out, dst_v, ones_v, idx_v, zero_v, deg_sp):
    c = lax.axis_index("c")
    s = lax.axis_index("s")
    wid = c * NS + s

    def _fill(i, _):
        zero_v[pl.ds(i * LANES, LANES)] = jnp.zeros((LANES,), jnp.float32)
        return 0

    lax.fori_loop(0, (NPAD // NS) // LANES, _fill, 0)
    pltpu.sync_copy(zero_v, deg_sp.at[pl.ds(s * (NPAD // NS), NPAD // NS)])

    def _fill1(i, _):
        ones_v[pl.ds(i * LANES, LANES)] = jnp.ones((LANES,), jnp.float32)
        return 0

    lax.fori_loop(0, K // LANES, _fill1, 0)
    pltpu.sync_copy(dst_hbm.at[wid], dst_v)
    plsc.subcore_barrier()

    def _chunk(j, _):
        def _cp(t, _):
            idx_v[pl.ds(t * LANES, LANES)] = dst_v[pl.ds(j * K + t * LANES, LANES)]
            return 0

        lax.fori_loop(0, K // LANES, _cp, 0)
        pltpu.sync_copy(ones_v, deg_sp.at[idx_v], add=True)
        return 0

    lax.fori_loop(0, 5120 // K, _chunk, 0)
    plsc.subcore_barrier()
    pltpu.sync_copy(deg_sp.at[pl.ds(s * (NPAD // NS), NPAD // NS)],
                    deg_out.at[c, pl.ds(s * (NPAD // NS), NPAD // NS)])


SEG = 1024
EPT = EPAD // NS
KE = 32
_ACC_WORDS = (HALF + NS) * D


@functools.partial(
    pl.kernel,
    out_type=jax.ShapeDtypeStruct((NPAD * D,), jnp.float32),
    mesh=_sc_mesh,
    scratch_types=[
        pltpu.VMEM((SEG,), jnp.int32),
        pltpu.VMEM((SEG,), jnp.int32),
        pltpu.VMEM((KE,), jnp.int32),
        pltpu.VMEM((KE,), jnp.int32),
        pltpu.VMEM((2, KE, D), jnp.float32),
        pltpu.VMEM((2, 2 * KE, K), jnp.int32),
        pltpu.VMEM_SHARED((_ACC_WORDS,), jnp.float32),
        pltpu.SemaphoreType.DMA,
        pltpu.SemaphoreType.DMA,
    ],
)
def _agg_kernel(zf_hbm, z2_hbm, src_hbm, dst_hbm, aggf_out,
                sseg_v, dseg_v, gidx_v, wb_v, rows_v, aidx_v, acc_sp, sem,
                gsem):
    c = lax.axis_index("c")
    s = lax.axis_index("s")
    nbase = c * HALF
    wslice = ROWS_PER_TILE * D
    ebase = s * EPT

    pltpu.sync_copy(zf_hbm.at[pl.ds((nbase + s * ROWS_PER_TILE) * D, wslice)],
                    acc_sp.at[pl.ds(s * ROWS_PER_TILE * D, wslice)])
    plsc.subcore_barrier()

    iota = lax.broadcasted_iota(jnp.int32, (LANES,), 0)

    def _segment(g, _):
        pltpu.sync_copy(src_hbm.at[pl.ds(ebase + g * SEG, SEG)], sseg_v)
        pltpu.sync_copy(dst_hbm.at[pl.ds(ebase + g * SEG, SEG)], dseg_v)

        def _chunk(j, _):
            def _prep(t, _):
                sl = pl.ds(j * KE + t * LANES, LANES)
                gidx_v[pl.ds(t * LANES, LANES)] = sseg_v[sl]
                d16 = dseg_v[sl]
                dl = d16 - nbase
                m = jnp.logical_and(dl >= 0, dl < HALF)
                wb_v[pl.ds(t * LANES, LANES)] = jnp.where(
                    m, dl, jnp.full((LANES,), HALF, jnp.int32) + s) * D
                return 0

            lax.fori_loop(0, KE // LANES, _prep, 0)
            b = jnp.bitwise_and(j, 1)
            gd = pltpu.async_copy(z2_hbm.at[gidx_v], rows_v.at[b], gsem)

            @pl.when(j >= 2)
            def _():
                pltpu.make_async_copy(z2_hbm.at[pl.ds(0, KE)],
                                      rows_v.at[b], sem).wait()

            def _build(e, _):
                wv = wb_v[pl.ds((e // LANES) * LANES, LANES)]
                wb_e = wv[jnp.full((LANES,), e % LANES, jnp.int32)]
                for q in range(D // LANES):
                    aidx_v[b, 2 * e + q // 8, pl.ds((q % 8) * LANES, LANES)] = (
                        wb_e + (q * LANES) + iota)
                return 0

            lax.fori_loop(0, KE, _build, 0)
            gd.wait()

            def _fire(e, _):
                pltpu.async_copy(rows_v.at[b, e, pl.ds(0, K)],
                                 acc_sp.at[aidx_v.at[b, 2 * e]], sem, add=True)
                pltpu.async_copy(rows_v.at[b, e, pl.ds(K, K)],
                                 acc_sp.at[aidx_v.at[b, 2 * e + 1]], sem,
                                 add=True)
                return 0

            lax.fori_loop(0, KE, _fire, 0)
            return 0

        lax.fori_loop(0, SEG // KE, _chunk, 0)
        pltpu.make_async_copy(z2_hbm.at[pl.ds(0, KE)], rows_v.at[0], sem).wait()
        pltpu.make_async_copy(z2_hbm.at[pl.ds(0, KE)], rows_v.at[1], sem).wait()
        return 0

    lax.fori_loop(0, EPT // SEG, _segment, 0)
    plsc.subcore_barrier()
    pltpu.sync_copy(acc_sp.at[pl.ds(s * ROWS_PER_TILE * D, wslice)],
                    aggf_out.at[pl.ds((nbase + s * ROWS_PER_TILE) * D, wslice)])


def _prep_body(x_ref, p_ref, deg_ref, score_ref, dinv_ref, z_ref):
    x = x_ref[...]
    p = p_ref[...]
    pn = jnp.sqrt(jnp.sum(p * p))
    sc = jnp.tanh(jax.lax.dot(x, p, preferred_element_type=jnp.float32) / pn)
    rows = lax.broadcasted_iota(jnp.int32, (NPAD, 1), 0)
    score_ref[...] = jnp.where(rows < N, sc, jnp.float32(-2.0))
    dg = deg_ref[...]
    deg = dg[:, 0:1] + dg[:, 1:2] + 1.0
    dinv = lax.rsqrt(deg)
    dinv_ref[...] = dinv
    z_ref[...] = x * dinv


def _prep(x_pad, p2, deg_t):
    return pl.pallas_call(
        _prep_body,
        out_shape=(
            jax.ShapeDtypeStruct((NPAD, 1), jnp.float32),
            jax.ShapeDtypeStruct((NPAD, 1), jnp.float32),
            jax.ShapeDtypeStruct((NPAD, D), jnp.float32),
        ),
    )(x_pad, p2, deg_t)


def _evolve_body(s_ref, x_ref, wih_ref, whh_ref, bih_ref, bhh_ref, w0_ref,
                 w_ref):
    S0 = s_ref[...]
    x = x_ref[...]
    flat = (lax.broadcasted_iota(jnp.int32, (80, 128), 0) * 128
            + lax.broadcasted_iota(jnp.int32, (80, 128), 1))
    lane = lax.broadcasted_iota(jnp.int32, (1, NPAD), 1)
    rowio = lax.broadcasted_iota(jnp.int32, (D, 1), 0)

    def _it(i, carry):
        S, Xt = carry
        m = jnp.max(S)
        idx = jnp.min(jnp.where(S == m, flat, jnp.int32(1 << 30)))
        onehot = (lane == idx).astype(jnp.float32)
        row = lax.dot_general(onehot, x, (((1,), (0,)), ((), ())),
                              preferred_element_type=jnp.float32) * m
        Xt = jnp.where(rowio == i, row, Xt)
        S = jnp.where(flat == idx, jnp.float32(-3e38), S)
        return S, Xt

    _, Xt = lax.fori_loop(0, D, _it, (S0, jnp.zeros((D, D), jnp.float32)))

    w0 = w0_ref[...]
    gi = lax.dot_general(Xt, wih_ref[...], (((1,), (1,)), ((), ())),
                         preferred_element_type=jnp.float32, precision=lax.Precision.HIGHEST) + bih_ref[...]
    gh = lax.dot_general(w0, whh_ref[...], (((1,), (1,)), ((), ())),
                         preferred_element_type=jnp.float32, precision=lax.Precision.HIGHEST) + bhh_ref[...]
    r = jax.nn.sigmoid(gi[:, 0:D] + gh[:, 0:D])
    z = jax.nn.sigmoid(gi[:, D:2 * D] + gh[:, D:2 * D])
    n = jnp.tanh(gi[:, 2 * D:3 * D] + r * gh[:, 2 * D:3 * D])
    w_ref[...] = (1.0 - z) * n + z * w0


def _evolve(score80, x_pad, wih, whh, bih2, bhh2, w0):
    return pl.pallas_call(
        _evolve_body,
        out_shape=jax.ShapeDtypeStruct((D, D), jnp.float32),
    )(score80, x_pad, wih, whh, bih2, bhh2, w0)


_BLK = 512


def _final_body(a_ref, dv_ref, w_ref, lw_ref, lb_ref, out_ref):
    t = jax.lax.dot(a_ref[...], w_ref[...],
                    preferred_element_type=jnp.float32, precision=lax.Precision.HIGHEST) * dv_ref[...]
    t = jnp.maximum(t, 0.0)
    out_ref[...] = lax.dot_general(t, lw_ref[...], (((1,), (1,)), ((), ())),
                                   preferred_element_type=jnp.float32, precision=lax.Precision.HIGHEST) + lb_ref[...]


def _final(agg, dinv, w, lin_w, lb2):
    return pl.pallas_call(
        _final_body,
        grid=(NPAD // _BLK,),
        in_specs=[
            pl.BlockSpec((_BLK, D), lambda i: (i, 0)),
            pl.BlockSpec((_BLK, 1), lambda i: (i, 0)),
            pl.BlockSpec((D, D), lambda i: (0, 0)),
            pl.BlockSpec((D, D), lambda i: (0, 0)),
            pl.BlockSpec((1, D), lambda i: (0, 0)),
        ],
        out_specs=pl.BlockSpec((_BLK, D), lambda i: (i, 0)),
        out_shape=jax.ShapeDtypeStruct((NPAD, D), jnp.float32),
    )(agg, dinv, w, lin_w, lb2)


def kernel(x, edge_index, pool_p, gru_W_ih, gru_W_hh, gru_b_ih, gru_b_hh,
           W0, lin_W, lin_b):
    src = edge_index[0]
    dst = edge_index[1]
    npad_rows = NPAD - N
    x_pad = jnp.concatenate(
        [x, jnp.zeros((npad_rows, D), jnp.float32)], axis=0)
    epad = EPAD - E
    src_p = jnp.concatenate([src, jnp.full((epad,), ZROW, jnp.int32)])
    dst_p = jnp.concatenate([dst, jnp.full((epad,), NPAD - 1, jnp.int32)])

    deg_parts = _degree_kernel(dst_p.reshape(NC * NS, EPAD // (NC * NS)))
    score, dinv, z = _prep(x_pad, pool_p.reshape(D, 1),
                           deg_parts.T.reshape(NPAD, NC))
    w = _evolve(score.reshape(80, 128), x_pad, gru_W_ih, gru_W_hh,
                gru_b_ih.reshape(1, 3 * D), gru_b_hh.reshape(1, 3 * D), W0)
    aggf = _agg_kernel(z.reshape(NPAD * D), z, src_p, dst_p)
    h = _final(aggf.reshape(NPAD, D), dinv, w, lin_W, lin_b.reshape(1, D))
    return h[:N]
